# baseline (speedup 1.0000x reference)
import jax
import jax.numpy as jnp
from jax import lax
from jax.experimental import pallas as pl
from jax.experimental.pallas import tpu as pltpu
from jax.experimental.pallas import tpu_sc as plsc

_B = 128
_V = 100000
_H = 200
_HP = 208
_NG = 16
_TC_ALL = 782
_TPH = 391
_T = 17
_NCH = _TPH // _T
_NEG = -3.0e38


def _sc_body(lg_hbm, gm_hbm, ids_hbm, pt_hbm, outv_hbm, outi_hbm,
             lg_v, gm_v, ids_v, pt_v, ov_v, oi_v,
             sl0, sg0, sl1, sg1):
    c = lax.axis_index("c")
    s = lax.axis_index("s")
    wid = s * 2 + c
    g = wid >> 1
    half = wid & 1
    lane = lax.broadcasted_iota(jnp.int32, (16,), 0)

    pltpu.sync_copy(ids_hbm.at[pl.ds(g * 8 * _HP, 8 * _HP)], ids_v)
    pltpu.sync_copy(pt_hbm.at[pl.ds(g * 8 * 32, 8 * 32)], pt_v)

    cb0 = half * _TPH

    def dma_start(ci, buf):
        col0 = (cb0 + ci * _T) * 128
        src_l = lg_hbm.at[g, pl.ds(0, 8), pl.ds(col0, _T * 128)]
        src_g = gm_hbm.at[g, pl.ds(0, 8), pl.ds(col0, _T * 128)]

        @pl.when(buf == 0)
        def _():
            pltpu.async_copy(src_l, lg_v.at[0], sl0)
            pltpu.async_copy(src_g, gm_v.at[0], sg0)

        @pl.when(buf == 1)
        def _():
            pltpu.async_copy(src_l, lg_v.at[1], sl1)
            pltpu.async_copy(src_g, gm_v.at[1], sg1)

    def dma_wait(ci, buf):
        col0 = (cb0 + ci * _T) * 128
        src_l = lg_hbm.at[g, pl.ds(0, 8), pl.ds(col0, _T * 128)]
        src_g = gm_hbm.at[g, pl.ds(0, 8), pl.ds(col0, _T * 128)]

        @pl.when(buf == 0)
        def _():
            pltpu.make_async_copy(src_l, lg_v.at[0], sl0).wait()
            pltpu.make_async_copy(src_g, gm_v.at[0], sg0).wait()

        @pl.when(buf == 1)
        def _():
            pltpu.make_async_copy(src_l, lg_v.at[1], sl1).wait()
            pltpu.make_async_copy(src_g, gm_v.at[1], sg1).wait()

    rowp = []
    for r8 in range(8):
        p16 = pt_v[pl.ds(r8 * 32, 16)]
        t16 = pt_v[pl.ds(r8 * 32 + 16, 16)]
        greedy = t16 < 1e-5
        teff = jnp.where(greedy, jnp.float32(1.0), t16)
        gsel = jnp.where(greedy, jnp.float32(0.0), jnp.float32(1.0))
        rowp.append((p16, teff, gsel))

    dma_start(0, 0)

    def do_chunk(ci, carry):
        buf = lax.rem(ci, 2)
        col0 = (cb0 + ci * _T) * 128
        dma_wait(ci, buf)

        @pl.when(ci + 1 < _NCH)
        def _():
            dma_start(ci + 1, 1 - buf)

        is_tail = (half == 1) & (ci == _NCH - 1)

        out = []
        for r8 in range(8):
            p16, teff, gsel = rowp[r8]
            bestv, besti = carry[2 * r8], carry[2 * r8 + 1]

            r16 = jnp.full((16,), r8, jnp.int32)
            bufv = jnp.full((16,), buf, jnp.int32)
            fixes = []
            for j in range(_HP // 16):
                idv = ids_v[pl.ds(r8 * _HP + j * 16, 16)]
                m = (idv >= col0) & (idv < col0 + _T * 128)
                loc = jnp.where(m, idv - col0, 0)
                val = plsc.load_gather(lg_v, [bufv, r16, loc], mask=m)
                fixes.append((loc, val - p16, m))
            for loc, val, m in fixes:
                plsc.store_scatter(lg_v, [bufv, r16, loc], val, mask=m)

            def step(k, sc, r8=r8, teff=teff, gsel=gsel, col0=col0,
                     buf=buf):
                bv, bi = sc
                lg16 = lg_v[buf, r8, pl.ds(k * 16, 16)]
                gm16 = gm_v[buf, r8, pl.ds(k * 16, 16)]
                idx = col0 + k * 16 + lane
                val = lg16 / teff + gm16 * gsel
                upd = val > bv
                return (jnp.where(upd, val, bv), jnp.where(upd, idx, bi))

            def step_masked(k, sc, r8=r8, teff=teff, gsel=gsel,
                            col0=col0, buf=buf):
                bv, bi = sc
                lg16 = lg_v[buf, r8, pl.ds(k * 16, 16)]
                gm16 = gm_v[buf, r8, pl.ds(k * 16, 16)]
                idx = col0 + k * 16 + lane
                val = lg16 / teff + gm16 * gsel
                val = jnp.where(idx < _V, val, jnp.float32(_NEG))
                upd = val > bv
                return (jnp.where(upd, val, bv), jnp.where(upd, idx, bi))

            def scan2(sc, stepfn, unroll):
                def body(mm, c4):
                    a0, i0, b0, j0 = c4
                    a0, i0 = stepfn(2 * mm, (a0, i0))
                    b0, j0 = stepfn(2 * mm + 1, (b0, j0))
                    return (a0, i0, b0, j0)
                a, i, b, j = lax.fori_loop(
                    0, _T * 4, body,
                    (sc[0], sc[1], jnp.full((16,), _NEG, jnp.float32),
                     jnp.zeros((16,), jnp.int32)), unroll=unroll)
                upd = (b > a) | ((b == a) & (j < i))
                return (jnp.where(upd, b, a), jnp.where(upd, j, i))

            bestv, besti = lax.cond(
                is_tail,
                lambda sc: scan2(sc, step_masked, 2),
                lambda sc: scan2(sc, step, 4),
                (bestv, besti))
            out.extend([bestv, besti])
        return tuple(out)

    init = []
    for _ in range(8):
        init.extend([jnp.full((16,), _NEG, jnp.float32),
                     jnp.zeros((16,), jnp.int32)])
    final = lax.fori_loop(0, _NCH, do_chunk, tuple(init))

    for r8 in range(8):
        bestv, besti = final[2 * r8], final[2 * r8 + 1]
        m = jnp.max(bestv)
        cand = jnp.where(bestv == m, besti, jnp.int32(2**30))
        mi = jnp.min(cand)
        ov_v[pl.ds(r8 * 16, 16)] = jnp.full((16,), m, jnp.float32)
        oi_v[pl.ds(r8 * 16, 16)] = jnp.full((16,), mi, jnp.int32)

    pltpu.sync_copy(ov_v, outv_hbm.at[pl.ds(wid * 128, 128)])
    pltpu.sync_copy(oi_v, outi_hbm.at[pl.ds(wid * 128, 128)])


_GUMBEL = None


def _gumbel():
    global _GUMBEL
    if _GUMBEL is None:
        def build():
            u = jax.random.uniform(jax.random.key(42), (_B, _V),
                                   dtype=jnp.float32, minval=1e-10,
                                   maxval=1.0)
            return (-jnp.log(-jnp.log(u))).reshape(_NG, 8, _V)
        _GUMBEL = jax.block_until_ready(jax.jit(build)())
    return _GUMBEL


def kernel(logits_next, presence_penalties, temperatures, token_ids):
    gm = _gumbel()
    lg = logits_next.reshape(_NG, 8, _V)
    ids = jnp.pad(token_ids.astype(jnp.int32), ((0, 0), (0, _HP - _H)),
                  constant_values=_V).reshape(-1)
    pt = jnp.broadcast_to(
        jnp.stack([presence_penalties, temperatures], axis=1)[:, :, None],
        (_B, 2, 16)).astype(jnp.float32).reshape(-1)

    mesh = plsc.VectorSubcoreMesh(core_axis_name="c", subcore_axis_name="s",
                                  num_cores=2, num_subcores=16)
    run = pl.kernel(
        _sc_body,
        out_type=(jax.ShapeDtypeStruct((32 * 128,), jnp.float32),
                  jax.ShapeDtypeStruct((32 * 128,), jnp.int32)),
        mesh=mesh,
        scratch_types=[
            pltpu.VMEM((2, 8, _T * 128), jnp.float32),
            pltpu.VMEM((2, 8, _T * 128), jnp.float32),
            pltpu.VMEM((8 * _HP,), jnp.int32),
            pltpu.VMEM((8 * 32,), jnp.float32),
            pltpu.VMEM((128,), jnp.float32),
            pltpu.VMEM((128,), jnp.int32),
            pltpu.SemaphoreType.DMA,
            pltpu.SemaphoreType.DMA,
            pltpu.SemaphoreType.DMA,
            pltpu.SemaphoreType.DMA,
        ],
        compiler_params=pltpu.CompilerParams(needs_layout_passes=False,
                                             use_tc_tiling_on_sc=True),
    )
    vals, idxs = run(lg, gm, ids, pt)
    v = vals.reshape(32, 8, 16)[:, :, 0]
    i = idxs.reshape(32, 8, 16)[:, :, 0]
    v0, v1 = v[0::2], v[1::2]
    i0, i1 = i[0::2], i[1::2]
    out = jnp.where(v1 > v0, i1, i0)
    return out.reshape(_B)


# speedup vs baseline: 1.0245x; 1.0245x over previous
import jax
import jax.numpy as jnp
from jax import lax
from jax.experimental import pallas as pl
from jax.experimental.pallas import tpu as pltpu
from jax.experimental.pallas import tpu_sc as plsc

_B = 128
_V = 100000
_H = 200
_HP = 208
_NG = 16
_TC_ALL = 782
_TPH = 391
_T = 23
_NCH = _TPH // _T
_NEG = -3.0e38


def _sc_body(lg_hbm, gm_hbm, ids_hbm, pt_hbm, outv_hbm, outi_hbm,
             lg_v, gm_v, ids_v, pt_v, ov_v, oi_v,
             sl0, sg0, sl1, sg1):
    c = lax.axis_index("c")
    s = lax.axis_index("s")
    wid = s * 2 + c
    g = wid >> 1
    half = wid & 1
    lane = lax.broadcasted_iota(jnp.int32, (16,), 0)

    pltpu.sync_copy(ids_hbm.at[pl.ds(g * 8 * _HP, 8 * _HP)], ids_v)
    pltpu.sync_copy(pt_hbm.at[pl.ds(g * 8 * 32, 8 * 32)], pt_v)

    cb0 = half * _TPH

    def dma_start(ci, buf):
        col0 = (cb0 + ci * _T) * 128
        src_l = lg_hbm.at[g, pl.ds(0, 8), pl.ds(col0, _T * 128)]
        src_g = gm_hbm.at[g, pl.ds(0, 8), pl.ds(col0, _T * 128)]

        @pl.when(buf == 0)
        def _():
            pltpu.async_copy(src_l, lg_v.at[0], sl0)
            pltpu.async_copy(src_g, gm_v.at[0], sg0)

        @pl.when(buf == 1)
        def _():
            pltpu.async_copy(src_l, lg_v.at[1], sl1)
            pltpu.async_copy(src_g, gm_v.at[1], sg1)

    def dma_wait(ci, buf):
        col0 = (cb0 + ci * _T) * 128
        src_l = lg_hbm.at[g, pl.ds(0, 8), pl.ds(col0, _T * 128)]
        src_g = gm_hbm.at[g, pl.ds(0, 8), pl.ds(col0, _T * 128)]

        @pl.when(buf == 0)
        def _():
            pltpu.make_async_copy(src_l, lg_v.at[0], sl0).wait()
            pltpu.make_async_copy(src_g, gm_v.at[0], sg0).wait()

        @pl.when(buf == 1)
        def _():
            pltpu.make_async_copy(src_l, lg_v.at[1], sl1).wait()
            pltpu.make_async_copy(src_g, gm_v.at[1], sg1).wait()

    rowp = []
    for r8 in range(8):
        p16 = pt_v[pl.ds(r8 * 32, 16)]
        t16 = pt_v[pl.ds(r8 * 32 + 16, 16)]
        greedy = t16 < 1e-5
        teff = jnp.where(greedy, jnp.float32(1.0), t16)
        gsel = jnp.where(greedy, jnp.float32(0.0), jnp.float32(1.0))
        rowp.append((p16, teff, gsel))

    dma_start(0, 0)

    def do_chunk(ci, carry):
        buf = lax.rem(ci, 2)
        col0 = (cb0 + ci * _T) * 128
        dma_wait(ci, buf)

        @pl.when(ci + 1 < _NCH)
        def _():
            dma_start(ci + 1, 1 - buf)

        is_tail = (half == 1) & (ci == _NCH - 1)

        out = []
        for r8 in range(8):
            p16, teff, gsel = rowp[r8]
            bestv, besti = carry[2 * r8], carry[2 * r8 + 1]

            r16 = jnp.full((16,), r8, jnp.int32)
            bufv = jnp.full((16,), buf, jnp.int32)
            fixes = []
            for j in range(_HP // 16):
                idv = ids_v[pl.ds(r8 * _HP + j * 16, 16)]
                m = (idv >= col0) & (idv < col0 + _T * 128)
                loc = jnp.where(m, idv - col0, 0)
                val = plsc.load_gather(lg_v, [bufv, r16, loc], mask=m)
                fixes.append((loc, val - p16, m))
            for loc, val, m in fixes:
                plsc.store_scatter(lg_v, [bufv, r16, loc], val, mask=m)

            def step(k, sc, r8=r8, teff=teff, gsel=gsel, col0=col0,
                     buf=buf):
                bv, bi = sc
                lg16 = lg_v[buf, r8, pl.ds(k * 16, 16)]
                gm16 = gm_v[buf, r8, pl.ds(k * 16, 16)]
                idx = col0 + k * 16 + lane
                val = lg16 / teff + gm16 * gsel
                upd = val > bv
                return (jnp.where(upd, val, bv), jnp.where(upd, idx, bi))

            def step_masked(k, sc, r8=r8, teff=teff, gsel=gsel,
                            col0=col0, buf=buf):
                bv, bi = sc
                lg16 = lg_v[buf, r8, pl.ds(k * 16, 16)]
                gm16 = gm_v[buf, r8, pl.ds(k * 16, 16)]
                idx = col0 + k * 16 + lane
                val = lg16 / teff + gm16 * gsel
                val = jnp.where(idx < _V, val, jnp.float32(_NEG))
                upd = val > bv
                return (jnp.where(upd, val, bv), jnp.where(upd, idx, bi))

            bestv, besti = lax.cond(
                is_tail,
                lambda sc: lax.fori_loop(0, _T * 8, step_masked, sc,
                                         unroll=4),
                lambda sc: lax.fori_loop(0, _T * 8, step, sc, unroll=8),
                (bestv, besti))
            out.extend([bestv, besti])
        return tuple(out)

    init = []
    for _ in range(8):
        init.extend([jnp.full((16,), _NEG, jnp.float32),
                     jnp.zeros((16,), jnp.int32)])
    final = lax.fori_loop(0, _NCH, do_chunk, tuple(init))

    for r8 in range(8):
        bestv, besti = final[2 * r8], final[2 * r8 + 1]
        m = jnp.max(bestv)
        cand = jnp.where(bestv == m, besti, jnp.int32(2**30))
        mi = jnp.min(cand)
        ov_v[pl.ds(r8 * 16, 16)] = jnp.full((16,), m, jnp.float32)
        oi_v[pl.ds(r8 * 16, 16)] = jnp.full((16,), mi, jnp.int32)

    pltpu.sync_copy(ov_v, outv_hbm.at[pl.ds(wid * 128, 128)])
    pltpu.sync_copy(oi_v, outi_hbm.at[pl.ds(wid * 128, 128)])


_GUMBEL = None


def _gumbel():
    global _GUMBEL
    if _GUMBEL is None:
        def build():
            u = jax.random.uniform(jax.random.key(42), (_B, _V),
                                   dtype=jnp.float32, minval=1e-10,
                                   maxval=1.0)
            return (-jnp.log(-jnp.log(u))).reshape(_NG, 8, _V)
        _GUMBEL = jax.block_until_ready(jax.jit(build)())
    return _GUMBEL


def kernel(logits_next, presence_penalties, temperatures, token_ids):
    gm = _gumbel()
    lg = logits_next.reshape(_NG, 8, _V)
    ids = jnp.pad(token_ids.astype(jnp.int32), ((0, 0), (0, _HP - _H)),
                  constant_values=_V).reshape(-1)
    pt = jnp.broadcast_to(
        jnp.stack([presence_penalties, temperatures], axis=1)[:, :, None],
        (_B, 2, 16)).astype(jnp.float32).reshape(-1)

    mesh = plsc.VectorSubcoreMesh(core_axis_name="c", subcore_axis_name="s",
                                  num_cores=2, num_subcores=16)
    run = pl.kernel(
        _sc_body,
        out_type=(jax.ShapeDtypeStruct((32 * 128,), jnp.float32),
                  jax.ShapeDtypeStruct((32 * 128,), jnp.int32)),
        mesh=mesh,
        scratch_types=[
            pltpu.VMEM((2, 8, _T * 128), jnp.float32),
            pltpu.VMEM((2, 8, _T * 128), jnp.float32),
            pltpu.VMEM((8 * _HP,), jnp.int32),
            pltpu.VMEM((8 * 32,), jnp.float32),
            pltpu.VMEM((128,), jnp.float32),
            pltpu.VMEM((128,), jnp.int32),
            pltpu.SemaphoreType.DMA,
            pltpu.SemaphoreType.DMA,
            pltpu.SemaphoreType.DMA,
            pltpu.SemaphoreType.DMA,
        ],
        compiler_params=pltpu.CompilerParams(needs_layout_passes=False,
                                             use_tc_tiling_on_sc=True),
    )
    vals, idxs = run(lg, gm, ids, pt)
    v = vals.reshape(32, 8, 16)[:, :, 0]
    i = idxs.reshape(32, 8, 16)[:, :, 0]
    v0, v1 = v[0::2], v[1::2]
    i0, i1 = i[0::2], i[1::2]
    out = jnp.where(v1 > v0, i1, i0)
    return out.reshape(_B)
